# baseline (speedup 1.0000x reference)
import functools

import jax
import jax.numpy as jnp
from jax import lax
from jax.experimental import pallas as pl
from jax.experimental.pallas import tpu as pltpu
from jax.experimental.pallas import tpu_sc as plsc

VOCAB = 1000000
N_EMBED = 64
BATCH = 16384
CTX = 20

NC = 2
NS = 16
NW = NC * NS

B_PER_W = BATCH // NW
IDX_PER_W = B_PER_W * CTX
GCHUNK = 128
N_GROWS = IDX_PER_W // GCHUNK
B_CHUNK = 64
I_CHUNK = B_CHUNK * CTX
G_PER_STEP = I_CHUNK // GCHUNK
N_STEPS = B_PER_W // B_CHUNK
LANES = 16
D_VECS = N_EMBED // LANES


def _sc_body(table_hbm, xr_hbm, out_hbm, idx_v, rows_v, out_v, sem):
    wid = lax.axis_index("s") * NC + lax.axis_index("c")
    pltpu.sync_copy(xr_hbm.at[wid], idx_v)

    inv_ctx = jnp.float32(1.0 / CTX)

    for g in range(N_STEPS):
        copies = [
            pltpu.async_copy(
                table_hbm.at[idx_v.at[g * G_PER_STEP + j]],
                rows_v.at[pl.ds(j * GCHUNK, GCHUNK)],
                sem,
            )
            for j in range(G_PER_STEP)
        ]
        for c in copies:
            c.wait()

        def acc_body(b, carry):
            base = b * CTX
            for d in range(D_VECS):
                acc = rows_v[base, pl.ds(LANES * d, LANES)]
                for c in range(1, CTX):
                    acc = acc + rows_v[base + c, pl.ds(LANES * d, LANES)]
                out_v[b, pl.ds(LANES * d, LANES)] = acc * inv_ctx
            return carry

        lax.fori_loop(0, B_CHUNK, acc_body, 0)
        pltpu.sync_copy(
            out_v, out_hbm.at[pl.ds(wid * B_PER_W + g * B_CHUNK, B_CHUNK)]
        )


@jax.jit
def _cbow_mean(x, table):
    xr = x.reshape(NW, N_GROWS, GCHUNK).astype(jnp.int32)
    mesh = plsc.VectorSubcoreMesh(core_axis_name="c", subcore_axis_name="s")
    k = pl.kernel(
        _sc_body,
        out_type=jax.ShapeDtypeStruct((BATCH, N_EMBED), jnp.float32),
        mesh=mesh,
        scratch_types=[
            pltpu.VMEM((N_GROWS, GCHUNK), jnp.int32),
            pltpu.VMEM((I_CHUNK, N_EMBED), jnp.float32),
            pltpu.VMEM((B_CHUNK, N_EMBED), jnp.float32),
            pltpu.SemaphoreType.DMA,
        ],
        compiler_params=pltpu.CompilerParams(use_tc_tiling_on_sc=False),
    )
    return k(table, xr)


def kernel(x, y, table):
    del y
    return _cbow_mean(x, table)
